# baseline (speedup 1.0000x reference)
import functools
import jax
import jax.numpy as jnp
from jax import lax
from jax.experimental import pallas as pl
from jax.experimental.pallas import tpu as pltpu
from jax.experimental.pallas import tpu_sc as plsc

B = 16384
D = 64
L = 16
CHUNK = 128

_info = plsc.get_sparse_core_info()
NC = _info.num_cores
NS = _info.num_subcores
NW = NC * NS
B_PER_W = B // NW
N_CHUNKS = B_PER_W // CHUNK


def _sc_kernel(center_hbm, context_hbm, in_hbm, out_hbm, dot_hbm,
               idx_c, idx_x, a_rows, b_rows, out_v, sem):
    wid = lax.axis_index("s") * NC + lax.axis_index("c")
    base = wid * B_PER_W

    pltpu.sync_copy(center_hbm.at[pl.ds(wid * N_CHUNKS, N_CHUNKS)], idx_c)
    pltpu.sync_copy(context_hbm.at[pl.ds(wid * N_CHUNKS, N_CHUNKS)], idx_x)

    copies = []
    for j in range(N_CHUNKS):
        copies.append(pltpu.async_copy(
            in_hbm.at[idx_c.at[j]], a_rows.at[pl.ds(j * CHUNK, CHUNK)], sem))
        copies.append(pltpu.async_copy(
            out_hbm.at[idx_x.at[j]], b_rows.at[pl.ds(j * CHUNK, CHUNK)], sem))
    for cp in copies:
        cp.wait()

    lane = lax.iota(jnp.int32, L)

    def body(g, _):
        rows = g * L + lane
        acc = None
        for d in range(D):
            cols = (lane + d) & (D - 1)
            a = plsc.load_gather(a_rows, [rows, cols])
            b = plsc.load_gather(b_rows, [rows, cols])
            acc = a * b if acc is None else acc + a * b
        out_v[pl.ds(g * L, L)] = acc
        return 0

    lax.fori_loop(0, B_PER_W // L, body, 0)

    pltpu.sync_copy(out_v, dot_hbm.at[pl.ds(base, B_PER_W)])


@jax.jit
def _word2vec_dot(center2d, context2d, in_table, out_table):
    mesh = plsc.VectorSubcoreMesh(core_axis_name="c", subcore_axis_name="s")
    k = functools.partial(
        pl.kernel,
        out_type=jax.ShapeDtypeStruct((B,), jnp.float32),
        mesh=mesh,
        compiler_params=pltpu.CompilerParams(
            needs_layout_passes=False, use_tc_tiling_on_sc=False),
        scratch_types=[
            pltpu.VMEM((N_CHUNKS, CHUNK), jnp.int32),
            pltpu.VMEM((N_CHUNKS, CHUNK), jnp.int32),
            pltpu.VMEM((B_PER_W, D), jnp.float32),
            pltpu.VMEM((B_PER_W, D), jnp.float32),
            pltpu.VMEM((B_PER_W,), jnp.float32),
            pltpu.SemaphoreType.DMA,
        ],
    )(_sc_kernel)
    return k(center2d, context2d, in_table, out_table)


def kernel(center, context, in_table, out_table):
    center2d = center.astype(jnp.int32).reshape(NW * N_CHUNKS, CHUNK)
    context2d = context.astype(jnp.int32).reshape(NW * N_CHUNKS, CHUNK)
    return _word2vec_dot(center2d, context2d, in_table, out_table)
